# baseline (speedup 1.0000x reference)
import jax
import jax.numpy as jnp
from jax.experimental import pallas as pl
from jax.experimental.pallas import tpu as pltpu

B = 1024
DATA_DIM = 128
COORD = 2
V = DATA_DIM // COORD
HIDDEN = 256
TDIM = 128
N = B * V

BB = 64
R = BB * V


def _silu(v):
    return 0.5 * v * (1.0 + jnp.tanh(0.5 * v))


def _spmm_rows(u):
    nxt = jnp.concatenate([u[BB:], u[:BB]], axis=0)
    prv = jnp.concatenate([u[-BB:], u[:-BB]], axis=0)
    return (u + nxt + prv) * jnp.float32(1.0 / 3.0)


def _body(coords, tcol, Wt, bt, W0, b0, W1, b1, W2, b2, W3, b3, Wres,
          Wh1, bh1, Wh2, bh2, out_ref):
    f32 = jnp.float32
    dot = lambda a, b: jnp.dot(a, b, preferred_element_type=f32)

    li = jax.lax.broadcasted_iota(jnp.int32, (1, TDIM), 1)
    lm = jnp.where(li >= TDIM // 2, li - TDIM // 2, li).astype(f32)
    freqs = jnp.exp(f32(-jnp.log(10000.0) / (TDIM // 2 - 1)) * lm)
    phase = jnp.where(li >= TDIM // 2, f32(jnp.pi / 2), f32(0.0))
    tf = tcol[...].astype(f32) * freqs + phase

    te = _silu(dot(jnp.sin(tf), Wt[...]) + bt[...])
    Wtp = W0[COORD:, :] + Wres[COORD:, :]
    c0 = dot(te, Wtp) + b0[...]
    c0t = jnp.broadcast_to(c0[None], (V, BB, HIDDEN)).reshape(R, HIDDEN)

    xT = jnp.swapaxes(coords[...], 0, 1)
    xv = xT.reshape(V, COORD, BB)
    c2 = jnp.swapaxes(xv, 1, 2).reshape(R, COORD)

    pre = dot(_spmm_rows(c2), W0[:COORD, :]) + dot(c2, Wres[:COORD, :])
    h = _silu(pre + c0t)

    third = f32(1.0 / 3.0)
    for W, b in ((W1, b1), (W2, b2), (W3, b3)):
        u = dot(h, W[...] * third)
        bb = b[...]
        top = _silu(u[R - BB:] + u[:BB] + u[BB:2 * BB] + bb + h[:BB])
        mid = _silu(u[:R - 2 * BB] + u[BB:R - BB] + u[2 * BB:] + bb
                    + h[BB:R - BB])
        bot = _silu(u[R - 2 * BB:R - BB] + u[R - BB:] + u[:BB] + bb
                    + h[R - BB:])
        h = jnp.concatenate([top, mid, bot], axis=0)

    g = _silu(dot(h, Wh1[...]) + bh1[...])
    res = dot(g, Wh2[...]) + bh2[...]
    rv = jnp.swapaxes(res.reshape(V, BB, COORD), 1, 2)
    out_ref[...] = jnp.swapaxes(rv.reshape(DATA_DIM, BB), 0, 1)


@jax.jit
def kernel(x, t, Wt, bt, W0, b0, W1, b1, W2, b2, W3, b3, Wres,
           Wh1, bh1, Wh2, bh2):
    grid = B // BB
    rep = lambda i: (0, 0)
    row = lambda v: v.reshape(1, -1)

    out = pl.pallas_call(
        _body,
        grid=(grid,),
        in_specs=[
            pl.BlockSpec((BB, DATA_DIM), lambda i: (i, 0)),
            pl.BlockSpec((BB, 1), lambda i: (i, 0)),
            pl.BlockSpec((TDIM, TDIM), rep),
            pl.BlockSpec((1, TDIM), rep),
            pl.BlockSpec((COORD + TDIM, HIDDEN), rep),
            pl.BlockSpec((1, HIDDEN), rep),
            pl.BlockSpec((HIDDEN, HIDDEN), rep),
            pl.BlockSpec((1, HIDDEN), rep),
            pl.BlockSpec((HIDDEN, HIDDEN), rep),
            pl.BlockSpec((1, HIDDEN), rep),
            pl.BlockSpec((HIDDEN, HIDDEN), rep),
            pl.BlockSpec((1, HIDDEN), rep),
            pl.BlockSpec((COORD + TDIM, HIDDEN), rep),
            pl.BlockSpec((HIDDEN, HIDDEN), rep),
            pl.BlockSpec((1, HIDDEN), rep),
            pl.BlockSpec((HIDDEN, COORD), rep),
            pl.BlockSpec((1, COORD), rep),
        ],
        out_specs=pl.BlockSpec((BB, DATA_DIM), lambda i: (i, 0)),
        out_shape=jax.ShapeDtypeStruct((B, DATA_DIM), jnp.float32),
        compiler_params=pltpu.CompilerParams(
            dimension_semantics=("parallel",)),
    )(x, t.reshape(B, 1), Wt, row(bt), W0, row(b0),
      W1, row(b1), W2, row(b2), W3, row(b3), Wres,
      Wh1, row(bh1), Wh2, row(bh2))

    return out


# speedup vs baseline: 1.0136x; 1.0136x over previous
import jax
import jax.numpy as jnp
from jax.experimental import pallas as pl
from jax.experimental.pallas import tpu as pltpu

B = 1024
DATA_DIM = 128
COORD = 2
V = DATA_DIM // COORD
HIDDEN = 256
TDIM = 128
N = B * V

BB = 128
R = BB * V


def _silu(v):
    return 0.5 * v * (1.0 + jnp.tanh(0.5 * v))


def _spmm_rows(u):
    nxt = jnp.concatenate([u[BB:], u[:BB]], axis=0)
    prv = jnp.concatenate([u[-BB:], u[:-BB]], axis=0)
    return (u + nxt + prv) * jnp.float32(1.0 / 3.0)


def _body(coords, tcol, Wt, bt, W0, b0, W1, b1, W2, b2, W3, b3, Wres,
          Wh1, bh1, Wh2, bh2, out_ref):
    f32 = jnp.float32
    dot = lambda a, b: jnp.dot(a, b, preferred_element_type=f32)

    li = jax.lax.broadcasted_iota(jnp.int32, (1, TDIM), 1)
    lm = jnp.where(li >= TDIM // 2, li - TDIM // 2, li).astype(f32)
    freqs = jnp.exp(f32(-jnp.log(10000.0) / (TDIM // 2 - 1)) * lm)
    phase = jnp.where(li >= TDIM // 2, f32(jnp.pi / 2), f32(0.0))
    tf = tcol[...].astype(f32) * freqs + phase

    te = _silu(dot(jnp.sin(tf), Wt[...]) + bt[...])
    Wtp = W0[COORD:, :] + Wres[COORD:, :]
    c0 = dot(te, Wtp) + b0[...]
    c0t = jnp.broadcast_to(c0[None], (V, BB, HIDDEN)).reshape(R, HIDDEN)

    xT = jnp.swapaxes(coords[...], 0, 1)
    xv = xT.reshape(V, COORD, BB)
    c2 = jnp.swapaxes(xv, 1, 2).reshape(R, COORD)

    pre = dot(_spmm_rows(c2), W0[:COORD, :]) + dot(c2, Wres[:COORD, :])
    h = _silu(pre + c0t)

    third = f32(1.0 / 3.0)
    for W, b in ((W1, b1), (W2, b2), (W3, b3)):
        u = dot(h, W[...] * third)
        bb = b[...]
        top = _silu(u[R - BB:] + u[:BB] + u[BB:2 * BB] + bb + h[:BB])
        mid = _silu(u[:R - 2 * BB] + u[BB:R - BB] + u[2 * BB:] + bb
                    + h[BB:R - BB])
        bot = _silu(u[R - 2 * BB:R - BB] + u[R - BB:] + u[:BB] + bb
                    + h[R - BB:])
        h = jnp.concatenate([top, mid, bot], axis=0)

    g = _silu(dot(h, Wh1[...]) + bh1[...])
    res = dot(g, Wh2[...]) + bh2[...]
    rv = jnp.swapaxes(res.reshape(V, BB, COORD), 1, 2)
    out_ref[...] = jnp.swapaxes(rv.reshape(DATA_DIM, BB), 0, 1)


@jax.jit
def kernel(x, t, Wt, bt, W0, b0, W1, b1, W2, b2, W3, b3, Wres,
           Wh1, bh1, Wh2, bh2):
    grid = B // BB
    rep = lambda i: (0, 0)
    row = lambda v: v.reshape(1, -1)

    out = pl.pallas_call(
        _body,
        grid=(grid,),
        in_specs=[
            pl.BlockSpec((BB, DATA_DIM), lambda i: (i, 0)),
            pl.BlockSpec((BB, 1), lambda i: (i, 0)),
            pl.BlockSpec((TDIM, TDIM), rep),
            pl.BlockSpec((1, TDIM), rep),
            pl.BlockSpec((COORD + TDIM, HIDDEN), rep),
            pl.BlockSpec((1, HIDDEN), rep),
            pl.BlockSpec((HIDDEN, HIDDEN), rep),
            pl.BlockSpec((1, HIDDEN), rep),
            pl.BlockSpec((HIDDEN, HIDDEN), rep),
            pl.BlockSpec((1, HIDDEN), rep),
            pl.BlockSpec((HIDDEN, HIDDEN), rep),
            pl.BlockSpec((1, HIDDEN), rep),
            pl.BlockSpec((COORD + TDIM, HIDDEN), rep),
            pl.BlockSpec((HIDDEN, HIDDEN), rep),
            pl.BlockSpec((1, HIDDEN), rep),
            pl.BlockSpec((HIDDEN, COORD), rep),
            pl.BlockSpec((1, COORD), rep),
        ],
        out_specs=pl.BlockSpec((BB, DATA_DIM), lambda i: (i, 0)),
        out_shape=jax.ShapeDtypeStruct((B, DATA_DIM), jnp.float32),
        compiler_params=pltpu.CompilerParams(
            dimension_semantics=("parallel",)),
    )(x, t.reshape(B, 1), Wt, row(bt), W0, row(b0),
      W1, row(b1), W2, row(b2), W3, row(b3), Wres,
      Wh1, row(bh1), Wh2, row(bh2))

    return out
